# baseline (speedup 1.0000x reference)
import functools

import jax
import jax.numpy as jnp
from jax import lax
from jax.experimental import pallas as pl
from jax.experimental.pallas import tpu as pltpu
from jax.experimental.pallas import tpu_sc as plsc

N = 10000
E = 320000
D = 128
NC = 2
NS = 16
NW = NC * NS
NP = 10240
K = 40
EP = E // NW
CPT = EP // K
SPT = NP // NS
RB = 1024
L = 16

_mesh = plsc.VectorSubcoreMesh(core_axis_name="c", subcore_axis_name="s")
_sc_params = pltpu.CompilerParams(needs_layout_passes=False)


def _deg_body(dst_hbm, deg_hbm, idx_v, cnt_v):
    c = lax.axis_index("c")
    s = lax.axis_index("s")
    wid = c * NS + s

    def zero(i, _):
        cnt_v[pl.ds(i * L, L)] = jnp.zeros((L,), jnp.float32)
        return 0

    lax.fori_loop(0, NP // L, zero, 0)

    pltpu.sync_copy(dst_hbm.at[pl.ds(wid * EP, EP)], idx_v)

    ones = jnp.ones((L,), jnp.float32)

    def count(i, _):
        idx = idx_v[pl.ds(i * L, L)]
        plsc.addupdate_scatter(cnt_v, [idx], ones)
        return 0

    lax.fori_loop(0, EP // L, count, 0)
    pltpu.sync_copy(cnt_v, deg_hbm.at[pl.ds(wid * NP, NP)])


@functools.partial(
    pl.kernel,
    out_type=jax.ShapeDtypeStruct((NW * NP,), jnp.float32),
    mesh=_mesh,
    compiler_params=_sc_params,
    scratch_types=[
        pltpu.VMEM((EP,), jnp.int32),
        pltpu.VMEM((NP,), jnp.float32),
    ],
)
def _deg_sc(dst_hbm, deg_hbm, idx_v, cnt_v):
    _deg_body(dst_hbm, deg_hbm, idx_v, cnt_v)


NBUF = 4
SLK = NBUF - 2
NGRP = (CPT + SLK + NBUF - 1) // NBUF


def _agg_body(hs_hbm, src_hbm, dst_hbm, zeros_hbm, acc0_hbm, acc1_hbm,
              src_v, dst_v, bufs, acc_sh, sg, ss):
    c = lax.axis_index("c")
    s = lax.axis_index("s")
    wid = c * NS + s

    pltpu.sync_copy(src_hbm.at[pl.ds(wid * EP, EP)], src_v)
    pltpu.sync_copy(dst_hbm.at[pl.ds(wid * EP, EP)], dst_v)

    rows = pl.ds(s * SPT, SPT)

    @pl.when(c == 0)
    def _():
        pltpu.sync_copy(hs_hbm.at[rows], acc_sh.at[rows])

    @pl.when(c == 1)
    def _():
        pltpu.sync_copy(zeros_hbm.at[rows], acc_sh.at[rows])

    plsc.subcore_barrier()

    def gather_start(j, b):
        pltpu.async_copy(hs_hbm.at[src_v.at[pl.ds(j * K, K)]], bufs[b], sg[b])

    def gather_wait(b):
        pltpu.make_async_copy(hs_hbm.at[pl.ds(0, K)], bufs[b], sg[b]).wait()

    def scatter_start(j, b):
        pltpu.async_copy(bufs[b], acc_sh.at[dst_v.at[pl.ds(j * K, K)]],
                         ss[b], add=True)

    def scatter_wait(b):
        pltpu.make_async_copy(bufs[b], acc_sh.at[pl.ds(0, K)], ss[b]).wait()

    for b in range(2):
        gather_start(b, b)

    def group(g, _):
        for b in range(NBUF):
            i = g * NBUF + b
            nb = (b + 2) % NBUF

            @pl.when(i < CPT)
            def _():
                gather_wait(b)
                scatter_start(i, b)

            @pl.when(jnp.logical_and(i >= SLK, i < CPT + SLK))
            def _():
                scatter_wait(nb)

            @pl.when(i + 2 < CPT)
            def _():
                gather_start(i + 2, nb)

        return 0

    lax.fori_loop(0, NGRP, group, 0)
    plsc.subcore_barrier()

    @pl.when(c == 0)
    def _():
        pltpu.sync_copy(acc_sh.at[rows], acc0_hbm.at[rows])

    @pl.when(c == 1)
    def _():
        pltpu.sync_copy(acc_sh.at[rows], acc1_hbm.at[rows])


@functools.partial(
    pl.kernel,
    out_type=(jax.ShapeDtypeStruct((NP, D), jnp.float32),
              jax.ShapeDtypeStruct((NP, D), jnp.float32)),
    mesh=_mesh,
    compiler_params=_sc_params,
    scratch_types=[
        pltpu.VMEM((EP,), jnp.int32),
        pltpu.VMEM((EP,), jnp.int32),
        [pltpu.VMEM((K, D), jnp.float32)] * NBUF,
        pltpu.VMEM_SHARED((NP, D), jnp.float32),
        [pltpu.SemaphoreType.DMA] * NBUF,
        [pltpu.SemaphoreType.DMA] * NBUF,
    ],
)
def _agg_sc(hs_hbm, src_hbm, dst_hbm, zeros_hbm, acc0_hbm, acc1_hbm,
            src_v, dst_v, bufs, acc_sh, sg, ss):
    _agg_body(hs_hbm, src_hbm, dst_hbm, zeros_hbm, acc0_hbm, acc1_hbm,
              src_v, dst_v, bufs, acc_sh, sg, ss)


def _degsum_body(deg_ref, o_ref):
    o_ref[...] = lax.rsqrt(jnp.sum(deg_ref[...], axis=0, keepdims=True) + 1.0)


_degsum_tc = pl.pallas_call(
    _degsum_body, grid=(1,),
    in_specs=[pl.BlockSpec((NW, NP), lambda i: (0, 0))],
    out_specs=pl.BlockSpec((1, NP), lambda i: (0, 0)),
    out_shape=jax.ShapeDtypeStruct((1, NP), jnp.float32))


def _mm(a, w):
    return lax.dot_general(a, w, (((1,), (0,)), ((), ())),
                           preferred_element_type=jnp.float32,
                           precision=lax.Precision.HIGHEST)


def _prologue_body(x_ref, w_ref, dis_ref, o_ref):
    o_ref[...] = _mm(x_ref[...], w_ref[...]) * dis_ref[...]


def _combine_body(a0_ref, a1_ref, dis_ref, b_ref, w_ref, o_ref):
    dis = dis_ref[...]
    t = (a0_ref[...] + a1_ref[...]) * dis + b_ref[...]
    xk = jnp.maximum(t, 0.0)
    o_ref[...] = _mm(xk, w_ref[...]) * dis


def _final_body(a0_ref, a1_ref, dis_ref, b_ref, o_ref):
    o_ref[...] = (a0_ref[...] + a1_ref[...]) * dis_ref[...] + b_ref[...]


_row_spec = pl.BlockSpec((RB, D), lambda i: (i, 0))
_dis_spec = pl.BlockSpec((RB, 1), lambda i: (i, 0))
_w_spec = pl.BlockSpec((D, D), lambda i: (0, 0))
_b_spec = pl.BlockSpec((1, D), lambda i: (0, 0))
_out_f32 = jax.ShapeDtypeStruct((NP, D), jnp.float32)
_grid = (NP // RB,)

_prologue_tc = pl.pallas_call(
    _prologue_body, grid=_grid,
    in_specs=[_row_spec, _w_spec, _dis_spec],
    out_specs=_row_spec, out_shape=_out_f32)

_combine_tc = pl.pallas_call(
    _combine_body, grid=_grid,
    in_specs=[_row_spec, _row_spec, _dis_spec, _b_spec, _w_spec],
    out_specs=_row_spec, out_shape=_out_f32)

_final_tc = pl.pallas_call(
    _final_body, grid=_grid,
    in_specs=[_row_spec, _row_spec, _dis_spec, _b_spec],
    out_specs=_row_spec, out_shape=_out_f32)


def kernel(x, edge_index, W1, b1, W2, b2, W3, b3):
    src = edge_index[0]
    dst = edge_index[1]
    x_p = jnp.pad(x, ((0, NP - N), (0, 0)))
    zeros = jnp.zeros((NP, D), jnp.float32)
    b1r = b1.reshape(1, D)
    b2r = b2.reshape(1, D)
    b3r = b3.reshape(1, D)

    deg = _deg_sc(dst)
    dis = _degsum_tc(deg.reshape(NW, NP)).reshape(NP, 1)

    hs1 = _prologue_tc(x_p, W1, dis)
    a0, a1 = _agg_sc(hs1, src, dst, zeros)
    hs2 = _combine_tc(a0, a1, dis, b1r, W2)
    a0, a1 = _agg_sc(hs2, src, dst, zeros)
    hs3 = _combine_tc(a0, a1, dis, b2r, W3)
    a0, a1 = _agg_sc(hs3, src, dst, zeros)
    return _final_tc(a0, a1, dis, b3r)[:N]


# speedup vs baseline: 1.2077x; 1.2077x over previous
import functools

import jax
import jax.numpy as jnp
from jax import lax
from jax.experimental import pallas as pl
from jax.experimental.pallas import tpu as pltpu
from jax.experimental.pallas import tpu_sc as plsc

N = 10000
E = 320000
D = 128
NC = 2
NS = 16
NW = NC * NS
NP = 10240
K = 64
CPT = 160
EP = K * CPT
EPAD = EP * NW - E
SPT = NP // NS
RB = 1024
L = 16

_mesh = plsc.VectorSubcoreMesh(core_axis_name="c", subcore_axis_name="s")
_sc_params = pltpu.CompilerParams(needs_layout_passes=False)


def _deg_body(dst_hbm, deg_hbm, idx_v, cnt_v):
    c = lax.axis_index("c")
    s = lax.axis_index("s")
    wid = c * NS + s

    def zero(i, _):
        cnt_v[pl.ds(i * L, L)] = jnp.zeros((L,), jnp.float32)
        return 0

    lax.fori_loop(0, NP // L, zero, 0)

    pltpu.sync_copy(dst_hbm.at[pl.ds(wid * EP, EP)], idx_v)

    ones = jnp.ones((L,), jnp.float32)

    def count(i, _):
        idx = idx_v[pl.ds(i * L, L)]
        plsc.addupdate_scatter(cnt_v, [idx], ones)
        return 0

    lax.fori_loop(0, EP // L, count, 0)
    pltpu.sync_copy(cnt_v, deg_hbm.at[pl.ds(wid * NP, NP)])


@functools.partial(
    pl.kernel,
    out_type=jax.ShapeDtypeStruct((NW * NP,), jnp.float32),
    mesh=_mesh,
    compiler_params=_sc_params,
    scratch_types=[
        pltpu.VMEM((EP,), jnp.int32),
        pltpu.VMEM((NP,), jnp.float32),
    ],
)
def _deg_sc(dst_hbm, deg_hbm, idx_v, cnt_v):
    _deg_body(dst_hbm, deg_hbm, idx_v, cnt_v)


NBUF = 3
SLK = NBUF - 2
NGRP = (CPT + SLK + NBUF - 1) // NBUF


def _agg_body(hs_hbm, src_hbm, dst_hbm, zeros_hbm, acc0_hbm, acc1_hbm,
              src_v, dst_v, bufs, acc_sh, sg, ss):
    c = lax.axis_index("c")
    s = lax.axis_index("s")
    wid = c * NS + s

    pltpu.sync_copy(src_hbm.at[pl.ds(wid * EP, EP)], src_v)
    pltpu.sync_copy(dst_hbm.at[pl.ds(wid * EP, EP)], dst_v)

    rows = pl.ds(s * SPT, SPT)

    @pl.when(c == 0)
    def _():
        pltpu.sync_copy(hs_hbm.at[rows], acc_sh.at[rows])

    @pl.when(c == 1)
    def _():
        pltpu.sync_copy(zeros_hbm.at[rows], acc_sh.at[rows])

    plsc.subcore_barrier()

    def gather_start(j, b):
        pltpu.async_copy(hs_hbm.at[src_v.at[pl.ds(j * K, K)]], bufs[b], sg[b])

    def gather_wait(b):
        pltpu.make_async_copy(hs_hbm.at[pl.ds(0, K)], bufs[b], sg[b]).wait()

    def scatter_start(j, b):
        pltpu.async_copy(bufs[b], acc_sh.at[dst_v.at[pl.ds(j * K, K)]],
                         ss[b], add=True)

    def scatter_wait(b):
        pltpu.make_async_copy(bufs[b], acc_sh.at[pl.ds(0, K)], ss[b]).wait()

    for b in range(2):
        gather_start(b, b)

    def group(g, _):
        for b in range(NBUF):
            i = g * NBUF + b
            nb = (b + 2) % NBUF

            @pl.when(i < CPT)
            def _():
                gather_wait(b)
                scatter_start(i, b)

            @pl.when(jnp.logical_and(i >= SLK, i < CPT + SLK))
            def _():
                scatter_wait(nb)

            @pl.when(i + 2 < CPT)
            def _():
                gather_start(i + 2, nb)

        return 0

    lax.fori_loop(0, NGRP, group, 0)
    plsc.subcore_barrier()

    @pl.when(c == 0)
    def _():
        pltpu.sync_copy(acc_sh.at[rows], acc0_hbm.at[rows])

    @pl.when(c == 1)
    def _():
        pltpu.sync_copy(acc_sh.at[rows], acc1_hbm.at[rows])


@functools.partial(
    pl.kernel,
    out_type=(jax.ShapeDtypeStruct((NP, D), jnp.float32),
              jax.ShapeDtypeStruct((NP, D), jnp.float32)),
    mesh=_mesh,
    compiler_params=_sc_params,
    scratch_types=[
        pltpu.VMEM((EP,), jnp.int32),
        pltpu.VMEM((EP,), jnp.int32),
        [pltpu.VMEM((K, D), jnp.float32)] * NBUF,
        pltpu.VMEM_SHARED((NP, D), jnp.float32),
        [pltpu.SemaphoreType.DMA] * NBUF,
        [pltpu.SemaphoreType.DMA] * NBUF,
    ],
)
def _agg_sc(hs_hbm, src_hbm, dst_hbm, zeros_hbm, acc0_hbm, acc1_hbm,
            src_v, dst_v, bufs, acc_sh, sg, ss):
    _agg_body(hs_hbm, src_hbm, dst_hbm, zeros_hbm, acc0_hbm, acc1_hbm,
              src_v, dst_v, bufs, acc_sh, sg, ss)


def _degsum_body(deg_ref, o_ref):
    o_ref[...] = lax.rsqrt(jnp.sum(deg_ref[...], axis=0, keepdims=True) + 1.0)


_degsum_tc = pl.pallas_call(
    _degsum_body, grid=(1,),
    in_specs=[pl.BlockSpec((NW, NP), lambda i: (0, 0))],
    out_specs=pl.BlockSpec((1, NP), lambda i: (0, 0)),
    out_shape=jax.ShapeDtypeStruct((1, NP), jnp.float32))


def _mm(a, w):
    return lax.dot_general(a, w, (((1,), (0,)), ((), ())),
                           preferred_element_type=jnp.float32,
                           precision=lax.Precision.HIGHEST)


def _prologue_body(x_ref, w_ref, dis_ref, o_ref):
    o_ref[...] = _mm(x_ref[...], w_ref[...]) * dis_ref[...]


def _combine_body(a0_ref, a1_ref, dis_ref, b_ref, w_ref, o_ref):
    dis = dis_ref[...]
    t = (a0_ref[...] + a1_ref[...]) * dis + b_ref[...]
    xk = jnp.maximum(t, 0.0)
    o_ref[...] = _mm(xk, w_ref[...]) * dis


def _final_body(a0_ref, a1_ref, dis_ref, b_ref, o_ref):
    o_ref[...] = (a0_ref[...] + a1_ref[...]) * dis_ref[...] + b_ref[...]


_row_spec = pl.BlockSpec((RB, D), lambda i: (i, 0))
_dis_spec = pl.BlockSpec((RB, 1), lambda i: (i, 0))
_w_spec = pl.BlockSpec((D, D), lambda i: (0, 0))
_b_spec = pl.BlockSpec((1, D), lambda i: (0, 0))
_out_f32 = jax.ShapeDtypeStruct((NP, D), jnp.float32)
_grid = (NP // RB,)

_prologue_tc = pl.pallas_call(
    _prologue_body, grid=_grid,
    in_specs=[_row_spec, _w_spec, _dis_spec],
    out_specs=_row_spec, out_shape=_out_f32)

_combine_tc = pl.pallas_call(
    _combine_body, grid=_grid,
    in_specs=[_row_spec, _row_spec, _dis_spec, _b_spec, _w_spec],
    out_specs=_row_spec, out_shape=_out_f32)

_final_tc = pl.pallas_call(
    _final_body, grid=_grid,
    in_specs=[_row_spec, _row_spec, _dis_spec, _b_spec],
    out_specs=_row_spec, out_shape=_out_f32)


def kernel(x, edge_index, W1, b1, W2, b2, W3, b3):
    pad_idx = (N + (jnp.arange(EPAD, dtype=jnp.int32) & 127))
    src = jnp.concatenate([edge_index[0], pad_idx])
    dst = jnp.concatenate([edge_index[1], pad_idx])
    x_p = jnp.pad(x, ((0, NP - N), (0, 0)))
    zeros = jnp.zeros((NP, D), jnp.float32)
    b1r = b1.reshape(1, D)
    b2r = b2.reshape(1, D)
    b3r = b3.reshape(1, D)

    deg = _deg_sc(dst)
    dis = _degsum_tc(deg.reshape(NW, NP)).reshape(NP, 1)

    hs1 = _prologue_tc(x_p, W1, dis)
    a0, a1 = _agg_sc(hs1, src, dst, zeros)
    hs2 = _combine_tc(a0, a1, dis, b1r, W2)
    a0, a1 = _agg_sc(hs2, src, dst, zeros)
    hs3 = _combine_tc(a0, a1, dis, b2r, W3)
    a0, a1 = _agg_sc(hs3, src, dst, zeros)
    return _final_tc(a0, a1, dis, b3r)[:N]


# speedup vs baseline: 1.2681x; 1.0501x over previous
import functools

import jax
import jax.numpy as jnp
from jax import lax
from jax.experimental import pallas as pl
from jax.experimental.pallas import tpu as pltpu
from jax.experimental.pallas import tpu_sc as plsc

N = 10000
E = 320000
D = 128
NC = 2
NS = 16
NW = NC * NS
NP = 10240
K = 80
CPT = 128
EP = K * CPT
EPAD = EP * NW - E
SPT = NP // NS
RB = 1024
L = 16

_mesh = plsc.VectorSubcoreMesh(core_axis_name="c", subcore_axis_name="s")
_sc_params = pltpu.CompilerParams(needs_layout_passes=False)


def _deg_body(dst_hbm, deg_hbm, idx_v, cnt_v):
    c = lax.axis_index("c")
    s = lax.axis_index("s")
    wid = c * NS + s

    def zero(i, _):
        cnt_v[pl.ds(i * L, L)] = jnp.zeros((L,), jnp.float32)
        return 0

    lax.fori_loop(0, NP // L, zero, 0)

    pltpu.sync_copy(dst_hbm.at[pl.ds(wid * EP, EP)], idx_v)

    ones = jnp.ones((L,), jnp.float32)

    def count(i, _):
        idx = idx_v[pl.ds(i * L, L)]
        plsc.addupdate_scatter(cnt_v, [idx], ones)
        return 0

    lax.fori_loop(0, EP // L, count, 0)
    pltpu.sync_copy(cnt_v, deg_hbm.at[pl.ds(wid * NP, NP)])


@functools.partial(
    pl.kernel,
    out_type=jax.ShapeDtypeStruct((NW * NP,), jnp.float32),
    mesh=_mesh,
    compiler_params=_sc_params,
    scratch_types=[
        pltpu.VMEM((EP,), jnp.int32),
        pltpu.VMEM((NP,), jnp.float32),
    ],
)
def _deg_sc(dst_hbm, deg_hbm, idx_v, cnt_v):
    _deg_body(dst_hbm, deg_hbm, idx_v, cnt_v)


NBUF = 3
SLK = NBUF - 2
NGRP = (CPT + SLK + NBUF - 1) // NBUF


def _agg_body(hs_hbm, pk_hbm, zeros_hbm, acc0_hbm, acc1_hbm,
              pk_v, sbufs, dbufs, bufs, acc_sh, sg, ss):
    c = lax.axis_index("c")
    s = lax.axis_index("s")
    wid = c * NS + s

    pltpu.sync_copy(pk_hbm.at[pl.ds(wid * EP, EP)], pk_v)

    rows = pl.ds(s * SPT, SPT)

    @pl.when(c == 0)
    def _():
        pltpu.sync_copy(hs_hbm.at[rows], acc_sh.at[rows])

    @pl.when(c == 1)
    def _():
        pltpu.sync_copy(zeros_hbm.at[rows], acc_sh.at[rows])

    plsc.subcore_barrier()

    def unpack(j, b):
        for q in range(K // L):
            p = pk_v[pl.ds(j * K + q * L, L)]
            sbufs[b][pl.ds(q * L, L)] = lax.shift_right_logical(p, 14)
            dbufs[b][pl.ds(q * L, L)] = lax.bitwise_and(p, 16383)

    def gather_start(j, b):
        unpack(j, b)
        pltpu.async_copy(hs_hbm.at[sbufs[b]], bufs[b], sg[b])

    def gather_wait(b):
        pltpu.make_async_copy(hs_hbm.at[pl.ds(0, K)], bufs[b], sg[b]).wait()

    def scatter_start(j, b):
        pltpu.async_copy(bufs[b], acc_sh.at[dbufs[b]], ss[b], add=True)

    def scatter_wait(b):
        pltpu.make_async_copy(bufs[b], acc_sh.at[pl.ds(0, K)], ss[b]).wait()

    for b in range(2):
        gather_start(b, b)

    def group(g, _):
        for b in range(NBUF):
            i = g * NBUF + b
            nb = (b + 2) % NBUF

            @pl.when(i < CPT)
            def _():
                gather_wait(b)
                scatter_start(i, b)

            @pl.when(jnp.logical_and(i >= SLK, i < CPT + SLK))
            def _():
                scatter_wait(nb)

            @pl.when(i + 2 < CPT)
            def _():
                gather_start(i + 2, nb)

        return 0

    lax.fori_loop(0, NGRP, group, 0)
    plsc.subcore_barrier()

    @pl.when(c == 0)
    def _():
        pltpu.sync_copy(acc_sh.at[rows], acc0_hbm.at[rows])

    @pl.when(c == 1)
    def _():
        pltpu.sync_copy(acc_sh.at[rows], acc1_hbm.at[rows])


@functools.partial(
    pl.kernel,
    out_type=(jax.ShapeDtypeStruct((NP, D), jnp.float32),
              jax.ShapeDtypeStruct((NP, D), jnp.float32)),
    mesh=_mesh,
    compiler_params=_sc_params,
    scratch_types=[
        pltpu.VMEM((EP,), jnp.int32),
        [pltpu.VMEM((K,), jnp.int32)] * NBUF,
        [pltpu.VMEM((K,), jnp.int32)] * NBUF,
        [pltpu.VMEM((K, D), jnp.float32)] * NBUF,
        pltpu.VMEM_SHARED((NP, D), jnp.float32),
        [pltpu.SemaphoreType.DMA] * NBUF,
        [pltpu.SemaphoreType.DMA] * NBUF,
    ],
)
def _agg_sc(hs_hbm, pk_hbm, zeros_hbm, acc0_hbm, acc1_hbm,
            pk_v, sbufs, dbufs, bufs, acc_sh, sg, ss):
    _agg_body(hs_hbm, pk_hbm, zeros_hbm, acc0_hbm, acc1_hbm,
              pk_v, sbufs, dbufs, bufs, acc_sh, sg, ss)


def _degsum_body(deg_ref, o_ref):
    o_ref[...] = lax.rsqrt(jnp.sum(deg_ref[...], axis=0, keepdims=True) + 1.0)


_degsum_tc = pl.pallas_call(
    _degsum_body, grid=(1,),
    in_specs=[pl.BlockSpec((NW, NP), lambda i: (0, 0))],
    out_specs=pl.BlockSpec((1, NP), lambda i: (0, 0)),
    out_shape=jax.ShapeDtypeStruct((1, NP), jnp.float32))


def _mm(a, w):
    return lax.dot_general(a, w, (((1,), (0,)), ((), ())),
                           preferred_element_type=jnp.float32,
                           precision=lax.Precision.HIGHEST)


def _prologue_body(x_ref, w_ref, dis_ref, o_ref):
    o_ref[...] = _mm(x_ref[...], w_ref[...]) * dis_ref[...]


def _combine_body(a0_ref, a1_ref, dis_ref, b_ref, w_ref, o_ref):
    dis = dis_ref[...]
    t = (a0_ref[...] + a1_ref[...]) * dis + b_ref[...]
    xk = jnp.maximum(t, 0.0)
    o_ref[...] = _mm(xk, w_ref[...]) * dis


def _final_body(a0_ref, a1_ref, dis_ref, b_ref, o_ref):
    o_ref[...] = (a0_ref[...] + a1_ref[...]) * dis_ref[...] + b_ref[...]


_row_spec = pl.BlockSpec((RB, D), lambda i: (i, 0))
_dis_spec = pl.BlockSpec((RB, 1), lambda i: (i, 0))
_w_spec = pl.BlockSpec((D, D), lambda i: (0, 0))
_b_spec = pl.BlockSpec((1, D), lambda i: (0, 0))
_out_f32 = jax.ShapeDtypeStruct((NP, D), jnp.float32)
_grid = (NP // RB,)

_prologue_tc = pl.pallas_call(
    _prologue_body, grid=_grid,
    in_specs=[_row_spec, _w_spec, _dis_spec],
    out_specs=_row_spec, out_shape=_out_f32)

_combine_tc = pl.pallas_call(
    _combine_body, grid=_grid,
    in_specs=[_row_spec, _row_spec, _dis_spec, _b_spec, _w_spec],
    out_specs=_row_spec, out_shape=_out_f32)

_final_tc = pl.pallas_call(
    _final_body, grid=_grid,
    in_specs=[_row_spec, _row_spec, _dis_spec, _b_spec],
    out_specs=_row_spec, out_shape=_out_f32)


def kernel(x, edge_index, W1, b1, W2, b2, W3, b3):
    pad_idx = (N + (jnp.arange(EPAD, dtype=jnp.int32) & 127))
    src = jnp.concatenate([edge_index[0], pad_idx])
    dst = jnp.concatenate([edge_index[1], pad_idx])
    packed = (src << 14) | dst
    x_p = jnp.pad(x, ((0, NP - N), (0, 0)))
    zeros = jnp.zeros((NP, D), jnp.float32)
    b1r = b1.reshape(1, D)
    b2r = b2.reshape(1, D)
    b3r = b3.reshape(1, D)

    deg = _deg_sc(dst)
    dis = _degsum_tc(deg.reshape(NW, NP)).reshape(NP, 1)

    hs1 = _prologue_tc(x_p, W1, dis)
    a0, a1 = _agg_sc(hs1, packed, zeros)
    hs2 = _combine_tc(a0, a1, dis, b1r, W2)
    a0, a1 = _agg_sc(hs2, packed, zeros)
    hs3 = _combine_tc(a0, a1, dis, b2r, W3)
    a0, a1 = _agg_sc(hs3, packed, zeros)
    return _final_tc(a0, a1, dis, b3r)[:N]


# speedup vs baseline: 1.2834x; 1.0120x over previous
import functools

import jax
import jax.numpy as jnp
from jax import lax
from jax.experimental import pallas as pl
from jax.experimental.pallas import tpu as pltpu
from jax.experimental.pallas import tpu_sc as plsc

N = 10000
E = 320000
D = 128
NC = 2
NS = 16
NW = NC * NS
NP = 10240
K = 80
CPT = 128
EP = K * CPT
EPAD = EP * NW - E
SPT = NP // NS
RB = 1024
L = 16

_mesh = plsc.VectorSubcoreMesh(core_axis_name="c", subcore_axis_name="s")
_sc_params = pltpu.CompilerParams(needs_layout_passes=False)


def _deg_body(dst_hbm, deg_hbm, idx_v, cnt_v):
    c = lax.axis_index("c")
    s = lax.axis_index("s")
    wid = c * NS + s

    def zero(i, _):
        cnt_v[pl.ds(i * L, L)] = jnp.zeros((L,), jnp.float32)
        return 0

    lax.fori_loop(0, NP // L, zero, 0)

    pltpu.sync_copy(dst_hbm.at[pl.ds(wid * EP, EP)], idx_v)

    ones = jnp.ones((L,), jnp.float32)

    def count(i, _):
        idx = idx_v[pl.ds(i * L, L)]
        plsc.addupdate_scatter(cnt_v, [idx], ones)
        return 0

    lax.fori_loop(0, EP // L, count, 0)
    pltpu.sync_copy(cnt_v, deg_hbm.at[pl.ds(wid * NP, NP)])


@functools.partial(
    pl.kernel,
    out_type=jax.ShapeDtypeStruct((NW * NP,), jnp.float32),
    mesh=_mesh,
    compiler_params=_sc_params,
    scratch_types=[
        pltpu.VMEM((EP,), jnp.int32),
        pltpu.VMEM((NP,), jnp.float32),
    ],
)
def _deg_sc(dst_hbm, deg_hbm, idx_v, cnt_v):
    _deg_body(dst_hbm, deg_hbm, idx_v, cnt_v)


NBUF = 3
SLK = NBUF - 2
NGRP = (CPT + SLK + NBUF - 1) // NBUF


def _agg_body(hs_hbm, pk_hbm, zeros_hbm, acc0_hbm, acc1_hbm,
              pk_v, sbufs, dbufs, bufs, acc_sh, sg, ss):
    c = lax.axis_index("c")
    s = lax.axis_index("s")
    wid = c * NS + s

    pltpu.sync_copy(pk_hbm.at[pl.ds(wid * EP, EP)], pk_v)

    rows = pl.ds(s * SPT, SPT)

    @pl.when(c == 0)
    def _():
        pltpu.sync_copy(hs_hbm.at[rows], acc_sh.at[rows])

    @pl.when(c == 1)
    def _():
        pltpu.sync_copy(zeros_hbm.at[rows], acc_sh.at[rows])

    plsc.subcore_barrier()

    def unpack(j, b):
        for q in range(K // L):
            p = pk_v[pl.ds(j * K + q * L, L)]
            sbufs[b][pl.ds(q * L, L)] = lax.shift_right_logical(p, 14)
            dbufs[b][pl.ds(q * L, L)] = lax.bitwise_and(p, 16383)

    def gather_start(j, b):
        unpack(j, b)
        pltpu.async_copy(hs_hbm.at[sbufs[b]], bufs[b], sg[b])

    def gather_wait(b):
        pltpu.make_async_copy(hs_hbm.at[pl.ds(0, K)], bufs[b], sg[b]).wait()

    def scatter_start(j, b):
        pltpu.async_copy(bufs[b], acc_sh.at[dbufs[b]], ss[b], add=True)

    def scatter_wait(b):
        pltpu.make_async_copy(bufs[b], acc_sh.at[pl.ds(0, K)], ss[b]).wait()

    for b in range(2):
        gather_start(b, b)

    def group(g, _):
        for b in range(NBUF):
            i = g * NBUF + b
            nb = (b + 2) % NBUF

            @pl.when(i < CPT)
            def _():
                gather_wait(b)
                scatter_start(i, b)

            @pl.when(jnp.logical_and(i >= SLK, i < CPT + SLK))
            def _():
                scatter_wait(nb)

            @pl.when(i + 2 < CPT)
            def _():
                gather_start(i + 2, nb)

        return 0

    lax.fori_loop(0, NGRP, group, 0)
    plsc.subcore_barrier()

    @pl.when(c == 0)
    def _():
        pltpu.sync_copy(acc_sh.at[rows], acc0_hbm.at[rows])

    @pl.when(c == 1)
    def _():
        pltpu.sync_copy(acc_sh.at[rows], acc1_hbm.at[rows])


@functools.partial(
    pl.kernel,
    out_type=(jax.ShapeDtypeStruct((NP, D), jnp.float32),
              jax.ShapeDtypeStruct((NP, D), jnp.float32)),
    mesh=_mesh,
    compiler_params=_sc_params,
    scratch_types=[
        pltpu.VMEM((EP,), jnp.int32),
        [pltpu.VMEM((K,), jnp.int32)] * NBUF,
        [pltpu.VMEM((K,), jnp.int32)] * NBUF,
        [pltpu.VMEM((K, D), jnp.float32)] * NBUF,
        pltpu.VMEM_SHARED((NP, D), jnp.float32),
        [pltpu.SemaphoreType.DMA] * NBUF,
        [pltpu.SemaphoreType.DMA] * NBUF,
    ],
)
def _agg_sc(hs_hbm, pk_hbm, zeros_hbm, acc0_hbm, acc1_hbm,
            pk_v, sbufs, dbufs, bufs, acc_sh, sg, ss):
    _agg_body(hs_hbm, pk_hbm, zeros_hbm, acc0_hbm, acc1_hbm,
              pk_v, sbufs, dbufs, bufs, acc_sh, sg, ss)


def _degsum_body(deg_ref, o_ref):
    o_ref[...] = lax.rsqrt(jnp.sum(deg_ref[...], axis=0, keepdims=True) + 1.0)


_degsum_tc = pl.pallas_call(
    _degsum_body, grid=(1,),
    in_specs=[pl.BlockSpec((NW, NP), lambda i: (0, 0))],
    out_specs=pl.BlockSpec((1, NP), lambda i: (0, 0)),
    out_shape=jax.ShapeDtypeStruct((1, NP), jnp.float32))


def _mm(a, w):
    return lax.dot_general(a, w, (((1,), (0,)), ((), ())),
                           preferred_element_type=jnp.float32,
                           precision=lax.Precision.HIGHEST)


def _prologue_body(x_ref, w_ref, dis_ref, o_ref):
    o_ref[...] = _mm(x_ref[...], w_ref[...]) * dis_ref[...]


def _combine_body(a0_ref, a1_ref, dis_ref, b_ref, w_ref, o_ref):
    dis = dis_ref[...]
    t = (a0_ref[...] + a1_ref[...]) * dis + b_ref[...]
    xk = jnp.maximum(t, 0.0)
    o_ref[...] = _mm(xk, w_ref[...]) * dis


def _final_body(a0_ref, a1_ref, dis_ref, b_ref, o_ref):
    o_ref[...] = (a0_ref[...] + a1_ref[...]) * dis_ref[...] + b_ref[...]


_row_spec = pl.BlockSpec((RB, D), lambda i: (i, 0))
_dis_spec = pl.BlockSpec((RB, 1), lambda i: (i, 0))
_w_spec = pl.BlockSpec((D, D), lambda i: (0, 0))
_b_spec = pl.BlockSpec((1, D), lambda i: (0, 0))
_out_f32 = jax.ShapeDtypeStruct((NP, D), jnp.float32)
_grid = (NP // RB,)

_prologue_tc = pl.pallas_call(
    _prologue_body, grid=_grid,
    in_specs=[_row_spec, _w_spec, _dis_spec],
    out_specs=_row_spec, out_shape=_out_f32)

_combine_tc = pl.pallas_call(
    _combine_body, grid=_grid,
    in_specs=[_row_spec, _row_spec, _dis_spec, _b_spec, _w_spec],
    out_specs=_row_spec, out_shape=_out_f32)

FRB = 1000

_final_tc = pl.pallas_call(
    _final_body, grid=(N // FRB,),
    in_specs=[pl.BlockSpec((FRB, D), lambda i: (i, 0)),
              pl.BlockSpec((FRB, D), lambda i: (i, 0)),
              pl.BlockSpec((FRB, 1), lambda i: (i, 0)),
              _b_spec],
    out_specs=pl.BlockSpec((FRB, D), lambda i: (i, 0)),
    out_shape=jax.ShapeDtypeStruct((N, D), jnp.float32))


def kernel(x, edge_index, W1, b1, W2, b2, W3, b3):
    pad_idx = (N + (jnp.arange(EPAD, dtype=jnp.int32) & 127))
    src = jnp.concatenate([edge_index[0], pad_idx])
    dst = jnp.concatenate([edge_index[1], pad_idx])
    packed = (src << 14) | dst
    x_p = jnp.pad(x, ((0, NP - N), (0, 0)))
    zeros = jnp.zeros((NP, D), jnp.float32)
    b1r = b1.reshape(1, D)
    b2r = b2.reshape(1, D)
    b3r = b3.reshape(1, D)

    deg = _deg_sc(dst)
    dis = _degsum_tc(deg.reshape(NW, NP)).reshape(NP, 1)

    hs1 = _prologue_tc(x_p, W1, dis)
    a0, a1 = _agg_sc(hs1, packed, zeros)
    hs2 = _combine_tc(a0, a1, dis, b1r, W2)
    a0, a1 = _agg_sc(hs2, packed, zeros)
    hs3 = _combine_tc(a0, a1, dis, b2r, W3)
    a0, a1 = _agg_sc(hs3, packed, zeros)
    return _final_tc(a0, a1, dis, b3r)
